# baseline (speedup 1.0000x reference)
import functools

import jax
import jax.numpy as jnp
from jax import lax
from jax.experimental import pallas as pl
from jax.experimental.pallas import tpu as pltpu
from jax.experimental.pallas import tpu_sc as plsc

_BATCH = 64


@functools.cache
def _build(B, V, D, NC, NS):
    NW = NC * NS
    b_per_w = B // NW
    L = 16

    mesh = plsc.VectorSubcoreMesh(core_axis_name="c", subcore_axis_name="s")

    @functools.partial(
        pl.kernel,
        mesh=mesh,
        out_type=jax.ShapeDtypeStruct((B // 8, 8, D), jnp.float32),
        scratch_types=[
            pltpu.VMEM((b_per_w,), jnp.int32),
            pltpu.VMEM((b_per_w // 8, 8, D), jnp.float32),
            pltpu.SemaphoreType.DMA,
        ],
    )
    def gather_kernel(idx_hbm, table_hbm, out_hbm, idx_v, out_v, sem):
        wid = lax.axis_index("s") * NC + lax.axis_index("c")
        base = wid * b_per_w
        pltpu.sync_copy(idx_hbm.at[pl.ds(base, b_per_w)], idx_v)

        def block(b):
            copies = []
            for v in range(_BATCH // L):
                iv = idx_v[pl.ds(b * _BATCH + v * L, L)]
                for j in range(L):
                    r = iv[j]
                    row = b * _BATCH + v * L + j
                    q = lax.shift_right_logical(row, 3)
                    rr = lax.bitwise_and(row, 7)
                    copies.append(pltpu.async_copy(
                        table_hbm.at[r], out_v.at[q, rr], sem))
            for c in copies:
                c.wait()

        pl.loop(0, b_per_w // _BATCH)(block)

        pltpu.sync_copy(out_v, out_hbm.at[pl.ds(wid * (b_per_w // 8),
                                                b_per_w // 8)])

    return gather_kernel


def kernel(subtask_indices, embedding_weight):
    idx = subtask_indices
    if idx.ndim > 1:
        idx = jnp.squeeze(idx, axis=-1)
    idx = idx.astype(jnp.int32)
    B = idx.shape[0]
    V, D = embedding_weight.shape

    info = plsc.get_sparse_core_info()
    NC, NS = info.num_cores, info.num_subcores

    out3 = _build(B, V, D, NC, NS)(idx, embedding_weight)
    return out3.reshape(B, D)


# speedup vs baseline: 1.0242x; 1.0242x over previous
import functools

import jax
import jax.numpy as jnp
from jax import lax
from jax.experimental import pallas as pl
from jax.experimental.pallas import tpu as pltpu
from jax.experimental.pallas import tpu_sc as plsc

_BATCH = 64


@functools.cache
def _build(B, V, D, NC, NS):
    NW = NC * NS
    b_per_w = B // NW
    L = 16

    mesh = plsc.VectorSubcoreMesh(core_axis_name="c", subcore_axis_name="s")

    @functools.partial(
        pl.kernel,
        mesh=mesh,
        out_type=jax.ShapeDtypeStruct((B // 8, 8, D), jnp.float32),
        scratch_types=[
            pltpu.VMEM((b_per_w,), jnp.int32),
            pltpu.VMEM((b_per_w // 8, 8, D), jnp.float32),
            pltpu.SemaphoreType.DMA,
        ],
    )
    def gather_kernel(idx_hbm, table_hbm, out_hbm, idx_v, out_v, sem):
        wid = lax.axis_index("s") * NC + lax.axis_index("c")
        base = wid * b_per_w
        pltpu.sync_copy(idx_hbm.at[pl.ds(base, b_per_w)], idx_v)

        def block(b):
            iv = idx_v[pl.ds(b * L, L)]
            for j in range(L):
                r = iv[j]
                row = b * L + j
                q = lax.shift_right_logical(row, 3)
                rr = lax.bitwise_and(row, 7)
                pltpu.async_copy(table_hbm.at[r], out_v.at[q, rr], sem)

        pl.loop(0, b_per_w // L)(block)
        pltpu.make_async_copy(out_hbm.at[pl.ds(0, b_per_w // 8)],
                              out_v, sem).wait()

        pltpu.sync_copy(out_v, out_hbm.at[pl.ds(wid * (b_per_w // 8),
                                                b_per_w // 8)])

    return gather_kernel


def kernel(subtask_indices, embedding_weight):
    idx = subtask_indices
    if idx.ndim > 1:
        idx = jnp.squeeze(idx, axis=-1)
    idx = idx.astype(jnp.int32)
    B = idx.shape[0]
    V, D = embedding_weight.shape

    info = plsc.get_sparse_core_info()
    NC, NS = info.num_cores, info.num_subcores

    out3 = _build(B, V, D, NC, NS)(idx, embedding_weight)
    return out3.reshape(B, D)
